# baseline (speedup 1.0000x reference)
import functools

import jax
import jax.numpy as jnp
from jax import lax
from jax.experimental import pallas as pl
from jax.experimental.pallas import tpu as pltpu
from jax.experimental.pallas import tpu_sc as plsc

_RES = 1024
_MIN_B = -2000.0
_MAX_B = 3000.0
_SCALE = (_RES - 1) / (_MAX_B - _MIN_B)

_MAGIC = 12582912.0
_CLO = _MAGIC
_CHI = _MAGIC + (_RES - 1)

_NC = 2
_NS = 16
_NW = _NC * _NS
_B, _N = 64, 4096
_TOTAL = _B * _N
_PPW = _TOTAL // _NW
_ROW = 128
_ROWS_PW = _PPW // _ROW
_VPR = _ROW // 16

_HOT_TR0 = 48
_HOT_BLKS = 16


def _quant(v):
  t = (v + jnp.float32(-_MIN_B)) * jnp.float32(_SCALE)
  z = t + jnp.float32(_MAGIC)
  z = jnp.minimum(jnp.maximum(z, jnp.float32(_CLO)), jnp.float32(_CHI))
  return plsc.bitcast(z, jnp.int32)


def _sc_kernel(x_hbm, w2_hbm, w1_hbm, out_hbm, xv, idxv, outv, hot, rows_smem,
               sem, hsem):
  sid = lax.axis_index("s")
  wid = sid * _NC + lax.axis_index("c")
  pltpu.make_async_copy(
      w2_hbm.at[pl.ds(_HOT_TR0, _HOT_BLKS), pl.ds(3 * 1024, 1024)],
      hot,
      hsem,
  ).start()
  pltpu.sync_copy(x_hbm.at[pl.ds(wid * (2 * _ROWS_PW), 2 * _ROWS_PW)], xv)
  pltpu.make_async_copy(
      w2_hbm.at[pl.ds(_HOT_TR0, _HOT_BLKS), pl.ds(3 * 1024, 1024)],
      hot,
      hsem,
  ).wait()

  def quant_pair(j, t):
    b = xv[2 * j, pl.ds(t * 16, 16)]
    d = xv[2 * j + 1, pl.ds(t * 16, 16)]
    return _quant(b), _quant(d)

  @plsc.parallel_loop(0, _ROWS_PW, step=1, unroll=4)
  def row(j):
    coldv = None
    for t in range(_VPR):
      qb, qd = quant_pair(j, t)
      local = ((qb & 127) << 7) | (qd & 127)
      outv[j, pl.ds(t * 16, 16)] = plsc.load_gather(
          hot, [local >> 10, local & 1023])
      isc = (((qb >> 7) & 7) != 3) | (((qd >> 7) & 7) != 3)
      coldv = isc if coldv is None else (coldv | isc)
    ncold = plsc.all_reduce_population_count(coldv)
    rows_smem[j] = jnp.max(ncold)

  def fire(j, nf):
    flag = (rows_smem[j] > 0).astype(jnp.int32)

    @pl.when(flag > 0)
    def _():
      for t in range(_VPR):
        qb, qd = quant_pair(j, t)
        flat = (((qb & 0x3F8) << 10) | ((qb & 7) << 7)
                | ((qd & 0x380) << 3) | (qd & 127))
        idxv[j, pl.ds(t * 16, 16)] = flat
      pltpu.make_async_copy(w1_hbm.at[idxv.at[j]], outv.at[j], sem).start()

    return nf + flag

  lax.fori_loop(0, _ROWS_PW, fire, 0)

  def drain(j, carry):
    @pl.when(rows_smem[j] > 0)
    def _():
      pltpu.make_async_copy(w1_hbm.at[idxv.at[j]], outv.at[j], sem).wait()

    return carry

  lax.fori_loop(0, _ROWS_PW, drain, 0)
  pltpu.sync_copy(outv, out_hbm.at[pl.ds(wid * _ROWS_PW, _ROWS_PW)])


@jax.jit
def kernel(x, w):
  mesh = plsc.VectorSubcoreMesh(core_axis_name="c", subcore_axis_name="s")
  run = functools.partial(
      pl.kernel,
      mesh=mesh,
      compiler_params=pltpu.CompilerParams(needs_layout_passes=False),
      out_type=jax.ShapeDtypeStruct((_TOTAL // _ROW, _ROW), jnp.float32),
      scratch_types=[
          pltpu.VMEM((2 * _ROWS_PW, _ROW), jnp.float32),
          pltpu.VMEM((_ROWS_PW, _ROW), jnp.int32),
          pltpu.VMEM((_ROWS_PW, _ROW), jnp.float32),
          pltpu.VMEM((_HOT_BLKS, 1024), jnp.float32),
          pltpu.SMEM((_ROWS_PW,), jnp.int32),
          pltpu.SemaphoreType.DMA,
          pltpu.SemaphoreType.DMA,
      ],
  )(_sc_kernel)
  x_lin = (x.transpose(0, 2, 1).reshape(_B, 2, _N // _ROW, _ROW)
           .transpose(0, 2, 1, 3).reshape(2 * _TOTAL // _ROW, _ROW))
  w_t = w.reshape(128, 8, 8, 128).transpose(0, 2, 1, 3)
  out = run(x_lin, w_t.reshape(128, 8192), w_t.reshape(-1))
  return out.reshape(_B, _N, 1)


# speedup vs baseline: 1.1360x; 1.1360x over previous
import functools

import jax
import jax.numpy as jnp
from jax import lax
from jax.experimental import pallas as pl
from jax.experimental.pallas import tpu as pltpu
from jax.experimental.pallas import tpu_sc as plsc

_RES = 1024
_MIN_B = -2000.0
_MAX_B = 3000.0
_SCALE = (_RES - 1) / (_MAX_B - _MIN_B)

_MAGIC = 12582912.0
_CLO = _MAGIC
_CHI = _MAGIC + (_RES - 1)

_NC = 2
_NS = 16
_NW = _NC * _NS
_B, _N = 64, 4096
_TOTAL = _B * _N
_PPW = _TOTAL // _NW
_ROW = 128
_ROWS_PW = _PPW // _ROW
_VPR = _ROW // 16

_HOT_TR0 = 48
_HOT_BLKS = 16


def _quant(v):
  t = (v + jnp.float32(-_MIN_B)) * jnp.float32(_SCALE)
  z = t + jnp.float32(_MAGIC)
  z = jnp.minimum(jnp.maximum(z, jnp.float32(_CLO)), jnp.float32(_CHI))
  return plsc.bitcast(z, jnp.int32)


def _sc_kernel(x_hbm, w1_hbm, out_hbm, xv, idxv, outv, hot, sem, hsem):
  sid = lax.axis_index("s")
  wid = sid * _NC + lax.axis_index("c")
  for i in range(_HOT_BLKS):
    pltpu.make_async_copy(
        w1_hbm.at[pl.ds(((_HOT_TR0 + i) * 8 + 3) * 1024, 1024)],
        hot.at[pl.ds(i * 1024, 1024)],
        hsem,
    ).start()
  pltpu.sync_copy(x_hbm.at[pl.ds(wid * (2 * _ROWS_PW), 2 * _ROWS_PW)], xv)
  for i in range(_HOT_BLKS):
    pltpu.make_async_copy(
        w1_hbm.at[pl.ds(((_HOT_TR0 + i) * 8 + 3) * 1024, 1024)],
        hot.at[pl.ds(i * 1024, 1024)],
        hsem,
    ).wait()

  def quant_pair(j, t):
    b = xv[2 * j, pl.ds(t * 16, 16)]
    d = xv[2 * j + 1, pl.ds(t * 16, 16)]
    return _quant(b), _quant(d)

  @plsc.parallel_loop(0, _ROWS_PW, step=1, unroll=2,
                      carry=jnp.zeros((16,), jnp.int32))
  def coldacc(j, acc):
    coldv = None
    for t in range(_VPR):
      qb, qd = quant_pair(j, t)
      local = ((qb & 127) << 7) | (qd & 127)
      outv[j, pl.ds(t * 16, 16)] = plsc.load_gather(hot, [local])
      isc = (((qb >> 7) & 7) ^ 3) | (((qd >> 7) & 7) ^ 3)
      coldv = isc if coldv is None else (coldv | isc)
    return acc | coldv

  any_cold = jnp.max(plsc.all_reduce_population_count(coldacc != 0))

  def fire(j, carry):
    for t in range(_VPR):
      qb, qd = quant_pair(j, t)
      flat = (((qb & 0x3F8) << 10) | ((qb & 7) << 7)
              | ((qd & 0x380) << 3) | (qd & 127))
      idxv[j, pl.ds(t * 16, 16)] = flat
    pltpu.make_async_copy(w1_hbm.at[idxv.at[j]], outv.at[j], sem).start()
    return carry

  def drain(j, carry):
    pltpu.make_async_copy(w1_hbm.at[idxv.at[j]], outv.at[j], sem).wait()
    return carry

  nrows = _ROWS_PW * (any_cold > 0).astype(jnp.int32)
  lax.fori_loop(0, nrows, fire, 0)
  lax.fori_loop(0, nrows, drain, 0)
  pltpu.sync_copy(outv, out_hbm.at[pl.ds(wid * _ROWS_PW, _ROWS_PW)])


@jax.jit
def kernel(x, w):
  mesh = plsc.VectorSubcoreMesh(core_axis_name="c", subcore_axis_name="s")
  run = functools.partial(
      pl.kernel,
      mesh=mesh,
      compiler_params=pltpu.CompilerParams(needs_layout_passes=False),
      out_type=jax.ShapeDtypeStruct((_TOTAL // _ROW, _ROW), jnp.float32),
      scratch_types=[
          pltpu.VMEM((2 * _ROWS_PW, _ROW), jnp.float32),
          pltpu.VMEM((_ROWS_PW, _ROW), jnp.int32),
          pltpu.VMEM((_ROWS_PW, _ROW), jnp.float32),
          pltpu.VMEM((_HOT_BLKS * 1024,), jnp.float32),
          pltpu.SemaphoreType.DMA,
          pltpu.SemaphoreType.DMA,
      ],
  )(_sc_kernel)
  x_lin = (x.transpose(0, 2, 1).reshape(_B, 2, _N // _ROW, _ROW)
           .transpose(0, 2, 1, 3).reshape(2 * _TOTAL // _ROW, _ROW))
  w_lin = w.reshape(128, 8, 8, 128).transpose(0, 2, 1, 3).reshape(-1)
  out = run(x_lin, w_lin)
  return out.reshape(_B, _N, 1)


# speedup vs baseline: 1.1540x; 1.0159x over previous
import functools

import jax
import jax.numpy as jnp
from jax import lax
from jax.experimental import pallas as pl
from jax.experimental.pallas import tpu as pltpu
from jax.experimental.pallas import tpu_sc as plsc

_RES = 1024
_MIN_B = -2000.0
_MAX_B = 3000.0
_SCALE = (_RES - 1) / (_MAX_B - _MIN_B)

_MAGIC = 12582912.0
_CLO = _MAGIC
_CHI = _MAGIC + (_RES - 1)

_NC = 2
_NS = 16
_NW = _NC * _NS
_B, _N = 64, 4096
_TOTAL = _B * _N
_PPW = _TOTAL // _NW
_ROW = 128
_ROWS_PW = _PPW // _ROW
_VPR = _ROW // 16

_HOT_TR0 = 48
_HOT_BLKS = 16


def _quant(v):
  t = (v + jnp.float32(-_MIN_B)) * jnp.float32(_SCALE)
  z = t + jnp.float32(_MAGIC)
  z = jnp.minimum(jnp.maximum(z, jnp.float32(_CLO)), jnp.float32(_CHI))
  return z, plsc.bitcast(z, jnp.int32)


def _sc_kernel(x_hbm, w1_hbm, out_hbm, xv, idxv, outv, hot, sem, hsem):
  sid = lax.axis_index("s")
  wid = sid * _NC + lax.axis_index("c")
  for i in range(_HOT_BLKS):
    pltpu.make_async_copy(
        w1_hbm.at[pl.ds(((_HOT_TR0 + i) * 8 + 3) * 1024, 1024)],
        hot.at[pl.ds(i * 1024, 1024)],
        hsem,
    ).start()
  pltpu.sync_copy(x_hbm.at[pl.ds(wid * (2 * _ROWS_PW), 2 * _ROWS_PW)], xv)
  for i in range(_HOT_BLKS):
    pltpu.make_async_copy(
        w1_hbm.at[pl.ds(((_HOT_TR0 + i) * 8 + 3) * 1024, 1024)],
        hot.at[pl.ds(i * 1024, 1024)],
        hsem,
    ).wait()

  def quant_pair(j, t):
    b = xv[2 * j, pl.ds(t * 16, 16)]
    d = xv[2 * j + 1, pl.ds(t * 16, 16)]
    zb, qb = _quant(b)
    zd, qd = _quant(d)
    return zb, zd, qb, qd

  @plsc.parallel_loop(0, _ROWS_PW, step=1, unroll=2,
                      carry=jnp.zeros((16,), jnp.bool_))
  def coldacc(j, acc):
    coldv = None
    for t in range(_VPR):
      zb, zd, qb, qd = quant_pair(j, t)
      local = ((qb & 127) << 7) | (qd & 127)
      outv[j, pl.ds(t * 16, 16)] = plsc.load_gather(hot, [local])
      isc = ((jnp.minimum(zb, zd) < jnp.float32(_CLO + 384))
             | (jnp.maximum(zb, zd) > jnp.float32(_CLO + 511)))
      coldv = isc if coldv is None else (coldv | isc)
    return acc | coldv

  any_cold = jnp.max(plsc.all_reduce_population_count(coldacc))

  def fire(j, carry):
    for t in range(_VPR):
      _, _, qb, qd = quant_pair(j, t)
      flat = (((qb & 0x3F8) << 10) | ((qb & 7) << 7)
              | ((qd & 0x380) << 3) | (qd & 127))
      idxv[j, pl.ds(t * 16, 16)] = flat
    pltpu.make_async_copy(w1_hbm.at[idxv.at[j]], outv.at[j], sem).start()
    return carry

  def drain(j, carry):
    pltpu.make_async_copy(w1_hbm.at[idxv.at[j]], outv.at[j], sem).wait()
    return carry

  nrows = _ROWS_PW * (any_cold > 0).astype(jnp.int32)
  lax.fori_loop(0, nrows, fire, 0)
  lax.fori_loop(0, nrows, drain, 0)
  pltpu.sync_copy(outv, out_hbm.at[pl.ds(wid * _ROWS_PW, _ROWS_PW)])


@jax.jit
def kernel(x, w):
  mesh = plsc.VectorSubcoreMesh(core_axis_name="c", subcore_axis_name="s")
  run = functools.partial(
      pl.kernel,
      mesh=mesh,
      compiler_params=pltpu.CompilerParams(needs_layout_passes=False),
      out_type=jax.ShapeDtypeStruct((_TOTAL // _ROW, _ROW), jnp.float32),
      scratch_types=[
          pltpu.VMEM((2 * _ROWS_PW, _ROW), jnp.float32),
          pltpu.VMEM((_ROWS_PW, _ROW), jnp.int32),
          pltpu.VMEM((_ROWS_PW, _ROW), jnp.float32),
          pltpu.VMEM((_HOT_BLKS * 1024,), jnp.float32),
          pltpu.SemaphoreType.DMA,
          pltpu.SemaphoreType.DMA,
      ],
  )(_sc_kernel)
  x_lin = (x.transpose(0, 2, 1).reshape(_B, 2, _N // _ROW, _ROW)
           .transpose(0, 2, 1, 3).reshape(2 * _TOTAL // _ROW, _ROW))
  w_lin = w.reshape(128, 8, 8, 128).transpose(0, 2, 1, 3).reshape(-1)
  out = run(x_lin, w_lin)
  return out.reshape(_B, _N, 1)
